# baseline (speedup 1.0000x reference)
import functools

import jax
import jax.numpy as jnp
from jax import lax
from jax.experimental import pallas as pl
from jax.experimental.pallas import tpu as pltpu
from jax.experimental.pallas import tpu_sc as plsc

BATCH = 16384
EMBED = 32
LANES = 16
NW = 32
NQ = 1000000
NBLK = (NQ + 127) // 128
TAILBLK = NBLK - 1
RBLK = NBLK // NW
WBLK = 8
NWAVES = 32
CAP = 784
WCAP = 144
IDCH = 4096
NMFLAT = 1000 * EMBED


@functools.cache
def _build_kernel():
    return functools.partial(
        pl.kernel,
        out_type=jax.ShapeDtypeStruct((BATCH,), jnp.float32),
        mesh=plsc.VectorSubcoreMesh(core_axis_name="c", subcore_axis_name="s"),
        compiler_params=pltpu.CompilerParams(
            needs_layout_passes=False, disable_bounds_checks=True),
        scratch_types=[
            pltpu.VMEM((2, EMBED, WBLK * 128), jnp.float32),
            pltpu.VMEM((272,), jnp.int32),
            pltpu.VMEM((NMFLAT,), jnp.float32),
            pltpu.VMEM((IDCH,), jnp.int32),
            pltpu.VMEM((IDCH,), jnp.int32),
            pltpu.VMEM((CAP,), jnp.int32),
            pltpu.VMEM((CAP,), jnp.int32),
            pltpu.VMEM((CAP,), jnp.int32),
            pltpu.VMEM((2, WCAP), jnp.int32),
            pltpu.VMEM((2, WCAP), jnp.int32),
            pltpu.VMEM((WCAP,), jnp.int32),
            pltpu.VMEM((WCAP,), jnp.int32),
            pltpu.VMEM((WCAP,), jnp.float32),
            pltpu.VMEM((WCAP,), jnp.float32),
            pltpu.SemaphoreType.DMA,
            pltpu.SemaphoreType.DMA,
            pltpu.SemaphoreType.DMA,
            pltpu.SemaphoreType.DMA,
        ],
    )(_mf_body)


def _mf_body(qids, mids, qtt, mtab, out, wave, bflag, mtv, qch, mch,
             myq, mym, myb, wq, wm, wba, wbb, wva, wvb, sema, semb, msem,
             ssem):
    wid = lax.axis_index("c") * 16 + lax.axis_index("s")
    rstart = wid * RBLK
    nblk = jnp.where(wid == NW - 1, NBLK - RBLK * (NW - 1), RBLK)
    rq0 = rstart * 128
    rq1 = (rstart + nblk) * 128

    mh = pltpu.async_copy(mtab, mtv, msem)

    iota = lax.iota(jnp.int32, LANES)
    ones = jnp.ones((LANES,), jnp.int32)
    neg = jnp.full((LANES,), -1, jnp.int32)
    zeros = jnp.zeros((LANES,), jnp.int32)
    for j in range(272 // LANES):
        bflag[pl.ds(j * LANES, LANES)] = zeros

    cnt = jnp.int32(0)
    for ch in range(BATCH // IDCH):
        pltpu.sync_copy(qids.at[pl.ds(ch * IDCH, IDCH)], qch)
        pltpu.sync_copy(mids.at[pl.ds(ch * IDCH, IDCH)], mch)

        def filt(g, c, ch=ch):
            sl = pl.ds(g * LANES, LANES)
            qv = qch[sl]
            m = (qv >= rq0) & (qv < rq1)
            plsc.store_compressed(myq.at[pl.ds(c, LANES)], qv, mask=m)
            plsc.store_compressed(mym.at[pl.ds(c, LANES)], mch[sl], mask=m)
            bb = ch * IDCH + g * LANES + iota
            plsc.store_compressed(myb.at[pl.ds(c, LANES)], bb, mask=m)
            plsc.store_scatter(
                bflag, [lax.shift_right_logical(qv, 7) - rstart], ones,
                mask=m)
            return c + plsc.all_reduce_population_count(m)[0]

        cnt = lax.fori_loop(0, IDCH // LANES, filt, cnt)

    ng = lax.shift_right_logical(cnt + LANES - 1, 4)
    mh.wait()

    sems = (sema, semb)
    bufs = ((wba, wva, sema), (wbb, wvb, semb))

    def fire(w):
        par = w & 1
        full = w * WBLK + WBLK <= nblk
        woff = pl.multiple_of((rstart + w * WBLK) * 128, 128)

        @pl.when(full & (par == 0))
        def _():
            pltpu.async_copy(qtt.at[:, pl.ds(woff, WBLK * 128)],
                             wave.at[0], sema)

        @pl.when(full & (par == 1))
        def _():
            pltpu.async_copy(qtt.at[:, pl.ds(woff, WBLK * 128)],
                             wave.at[1], semb)

        fv = bflag[pl.ds(w * WBLK, LANES)]
        n = jnp.int32(0)
        for i in range(WBLK):
            bl = w * WBLK + i
            blk = rstart + bl
            off = pl.multiple_of(blk * 128, 128)
            live = jnp.logical_not(full) & (bl < nblk) & (fv[i] > 0)

            @pl.when(live & (par == 0))
            def _():
                pltpu.async_copy(qtt.at[:, pl.ds(off, 128)],
                                 wave.at[0, :, pl.ds(i * 128, 128)], sema)

            @pl.when(live & (par == 1))
            def _():
                pltpu.async_copy(qtt.at[:, pl.ds(off, 128)],
                                 wave.at[1, :, pl.ds(i * 128, 128)], semb)

            n = n + jnp.where(live, 1, 0)
        return n

    def run_wave(w, k, buf):
        wb, wv, s = bufs[buf]
        full = w * WBLK + WBLK <= nblk

        @pl.when(full)
        def _():
            pltpu.make_async_copy(qtt.at[:, pl.ds(0, WBLK * 128)],
                                  wave.at[buf], sems[buf]).wait()

        drain = pltpu.make_async_copy(qtt.at[:, pl.ds(0, 128)],
                                      wave.at[buf, :, pl.ds(0, 128)],
                                      sems[buf])
        nf = jnp.int32(0)
        fv = bflag[pl.ds(w * WBLK, LANES)]
        for i in range(WBLK):
            nf = nf + jnp.where(jnp.logical_not(full) &
                                (w * WBLK + i < nblk) & (fv[i] > 0), 1, 0)
        lax.fori_loop(0, nf, lambda i, c: (drain.wait(), c)[1], 0)

        @pl.when(k > 0)
        def _():
            pltpu.make_async_copy(
                wv, out.at[plsc.Indices(wb, ignored_value=-1)], ssem).wait()

        for j in range(WCAP // LANES):
            wb[pl.ds(j * LANES, LANES)] = neg

        wq0 = (rstart + w * WBLK) * 128

        def wfilt(j, c):
            sl = pl.ds(j * LANES, LANES)
            qv = myq[sl]
            m = ((qv >= wq0) & (qv < wq0 + WBLK * 128) &
                 (j * LANES + iota < cnt))
            plsc.store_compressed(wq.at[buf, pl.ds(c, LANES)], qv, mask=m)
            plsc.store_compressed(wm.at[buf, pl.ds(c, LANES)], mym[sl], mask=m)
            plsc.store_compressed(wb.at[pl.ds(c, LANES)], myb[sl], mask=m)
            return c + plsc.all_reduce_population_count(m)[0]

        wcnt = lax.fori_loop(0, ng, wfilt, jnp.int32(0))

        def dot(u, c):
            sl = pl.ds(u * LANES, LANES)
            um = u * LANES + iota < wcnt
            qloc = wq[buf, sl] - wq0
            mbase = wm[buf, sl] * EMBED
            acc = jnp.zeros((LANES,), jnp.float32)
            qwave = wave.at[buf]
            for d in range(EMBED):
                qval = plsc.load_gather(
                    qwave, [jnp.full((LANES,), d, jnp.int32), qloc],
                    mask=um)
                mval = plsc.load_gather(mtv, [mbase + d], mask=um)
                acc = acc + qval * mval
            wv[sl] = acc
            return c

        lax.fori_loop(0, lax.shift_right_logical(wcnt + LANES - 1, 4), dot, 0)
        pltpu.async_copy(wv, out.at[plsc.Indices(wb, ignored_value=-1)], ssem)

    fire(jnp.int32(0))

    def pair(k, c):
        fire(2 * k + 1)
        run_wave(2 * k, k, 0)
        fire(2 * k + 2)
        run_wave(2 * k + 1, k, 1)
        return c

    lax.fori_loop(0, NWAVES // 2, pair, 0)

    for buf in range(2):
        wb, wv, s = bufs[buf]
        pltpu.make_async_copy(
            wv, out.at[plsc.Indices(wb, ignored_value=-1)], ssem).wait()


@jax.jit
def kernel(query_ids, model_ids, query_table, model_table):
    return _build_kernel()(query_ids.astype(jnp.int32),
                           model_ids.astype(jnp.int32),
                           query_table.T,
                           model_table.reshape(NMFLAT))


# speedup vs baseline: 1.1036x; 1.1036x over previous
import functools

import jax
import jax.numpy as jnp
from jax import lax
from jax.experimental import pallas as pl
from jax.experimental.pallas import tpu as pltpu
from jax.experimental.pallas import tpu_sc as plsc

BATCH = 16384
EMBED = 32
LANES = 16
NW = 32
NQ = 1000000
NBLK = (NQ + 127) // 128
TAILBLK = NBLK - 1
RBLK = NBLK // NW
WBLK = 8
NWAVES = 32
CAP = 784
WCAP = 144
IDCH = 4096
NMFLAT = 1000 * EMBED


@functools.cache
def _build_kernel():
    return functools.partial(
        pl.kernel,
        out_type=jax.ShapeDtypeStruct((BATCH,), jnp.float32),
        mesh=plsc.VectorSubcoreMesh(core_axis_name="c", subcore_axis_name="s"),
        compiler_params=pltpu.CompilerParams(
            needs_layout_passes=False, disable_bounds_checks=True),
        scratch_types=[
            pltpu.VMEM((2, WBLK, EMBED, 128), jnp.float32),
            pltpu.VMEM((288,), jnp.int32),
            pltpu.VMEM((NMFLAT,), jnp.float32),
            pltpu.VMEM((2, IDCH), jnp.int32),
            pltpu.VMEM((2, IDCH), jnp.int32),
            pltpu.VMEM((CAP,), jnp.int32),
            pltpu.VMEM((CAP,), jnp.int32),
            pltpu.VMEM((CAP,), jnp.int32),
            pltpu.VMEM((2, WCAP), jnp.int32),
            pltpu.VMEM((2, WCAP), jnp.int32),
            pltpu.VMEM((WCAP,), jnp.int32),
            pltpu.VMEM((WCAP,), jnp.int32),
            pltpu.VMEM((WCAP,), jnp.float32),
            pltpu.VMEM((WCAP,), jnp.float32),
            pltpu.SemaphoreType.DMA,
            pltpu.SemaphoreType.DMA,
            pltpu.SemaphoreType.DMA,
            pltpu.SemaphoreType.DMA,
            pltpu.SemaphoreType.DMA,
        ],
    )(_mf_body)


def _mf_body(qids, mids, qtt, mtab, out, wave, bflag, mtv, qch, mch,
             myq, mym, myb, wq, wm, wba, wbb, wva, wvb, sema, semb, msem,
             isem, ssem):
    wid = lax.axis_index("c") * 16 + lax.axis_index("s")
    rstart = wid * RBLK
    nblk = jnp.where(wid == NW - 1, NBLK - RBLK * (NW - 1), RBLK)
    rq0 = rstart * 128
    rq1 = (rstart + nblk) * 128

    mh = pltpu.async_copy(mtab, mtv, msem)

    iota = lax.iota(jnp.int32, LANES)
    ones = jnp.ones((LANES,), jnp.int32)
    neg = jnp.full((LANES,), -1, jnp.int32)
    zeros = jnp.zeros((LANES,), jnp.int32)
    for j in range(288 // LANES):
        bflag[pl.ds(j * LANES, LANES)] = zeros

    def stage(ch):
        p = ch % 2
        h1 = pltpu.async_copy(qids.at[pl.ds(ch * IDCH, IDCH)], qch.at[p],
                              isem)
        h2 = pltpu.async_copy(mids.at[pl.ds(ch * IDCH, IDCH)], mch.at[p],
                              isem)
        return (h1, h2)

    hs = [stage(0), stage(1)]

    for w0 in range(2):
        for i in range(WBLK):
            bl = w0 * WBLK + i
            off = pl.multiple_of((rstart + bl) * 128, 128)

            @pl.when(bl < nblk)
            def _(w0=w0, i=i, off=off):
                pltpu.async_copy(qtt.at[:, pl.ds(off, 128)],
                                 wave.at[w0, i], (sema, semb)[w0])

    cnt = jnp.int32(0)
    for ch in range(BATCH // IDCH):
        for h in hs[ch]:
            h.wait()
        p = ch % 2

        def filt(g, c, ch=ch, p=p):
            sl = pl.ds(g * LANES, LANES)
            qv = qch[p, sl]
            m = (qv >= rq0) & (qv < rq1)
            plsc.store_compressed(myq.at[pl.ds(c, LANES)], qv, mask=m)
            plsc.store_compressed(mym.at[pl.ds(c, LANES)], mch[p, sl],
                                  mask=m)
            bb = ch * IDCH + g * LANES + iota
            plsc.store_compressed(myb.at[pl.ds(c, LANES)], bb, mask=m)
            plsc.store_scatter(
                bflag, [lax.shift_right_logical(qv, 7) - rstart], ones,
                mask=m)
            return c + plsc.all_reduce_population_count(m)[0]

        cnt = lax.fori_loop(0, IDCH // LANES, filt, cnt)
        if ch + 2 < BATCH // IDCH:
            hs.append(stage(ch + 2))

    ng = lax.shift_right_logical(cnt + LANES - 1, 4)
    mh.wait()

    sems = (sema, semb)
    bufs = ((wba, wva, sema), (wbb, wvb, semb))

    def fire(w):
        n = jnp.int32(0)
        fv = bflag[pl.ds(w * WBLK, LANES)]
        for i in range(WBLK):
            bl = w * WBLK + i
            blk = rstart + bl
            off = pl.multiple_of(blk * 128, 128)
            live = (bl < nblk) & (fv[i] > 0)
            par = w & 1

            @pl.when(live & (par == 0))
            def _():
                pltpu.async_copy(qtt.at[:, pl.ds(off, 128)],
                                 wave.at[0, i], sema)

            @pl.when(live & (par == 1))
            def _():
                pltpu.async_copy(qtt.at[:, pl.ds(off, 128)],
                                 wave.at[1, i], semb)

            n = n + jnp.where(live, 1, 0)
        return n

    def run_wave(w, k, buf, noskip=False):
        wb, wv, s = bufs[buf]
        drain = pltpu.make_async_copy(qtt.at[:, pl.ds(0, 128)],
                                      wave.at[buf, 0], sems[buf])
        nf = jnp.int32(0)
        fv = bflag[pl.ds(w * WBLK, LANES)]
        for i in range(WBLK):
            live = w * WBLK + i < nblk
            if not noskip:
                live = live & (fv[i] > 0)
            nf = nf + jnp.where(live, 1, 0)
        lax.fori_loop(0, nf, lambda i, c: (drain.wait(), c)[1], 0)

        @pl.when(k > 0)
        def _():
            pltpu.make_async_copy(
                wv, out.at[plsc.Indices(wb, ignored_value=-1)], ssem).wait()

        for j in range(WCAP // LANES):
            wb[pl.ds(j * LANES, LANES)] = neg

        wq0 = (rstart + w * WBLK) * 128

        def wfilt(j, c):
            sl = pl.ds(j * LANES, LANES)
            qv = myq[sl]
            m = ((qv >= wq0) & (qv < wq0 + WBLK * 128) &
                 (j * LANES + iota < cnt))
            plsc.store_compressed(wq.at[buf, pl.ds(c, LANES)], qv, mask=m)
            plsc.store_compressed(wm.at[buf, pl.ds(c, LANES)], mym[sl], mask=m)
            plsc.store_compressed(wb.at[pl.ds(c, LANES)], myb[sl], mask=m)
            return c + plsc.all_reduce_population_count(m)[0]

        wcnt = lax.fori_loop(0, ng, wfilt, jnp.int32(0))

        def dot(u, c):
            sl = pl.ds(u * LANES, LANES)
            um = u * LANES + iota < wcnt
            qloc = wq[buf, sl] - wq0
            slot = lax.shift_right_logical(qloc, 7)
            col = qloc & 127
            mbase = wm[buf, sl] * EMBED
            acc = jnp.zeros((LANES,), jnp.float32)
            qwave = wave.at[buf]
            for d in range(EMBED):
                qval = plsc.load_gather(
                    qwave, [slot, jnp.full((LANES,), d, jnp.int32), col],
                    mask=um)
                mval = plsc.load_gather(mtv, [mbase + d], mask=um)
                acc = acc + qval * mval
            wv[sl] = acc
            return c

        lax.fori_loop(0, lax.shift_right_logical(wcnt + LANES - 1, 4), dot, 0)
        pltpu.async_copy(wv, out.at[plsc.Indices(wb, ignored_value=-1)], ssem)

    run_wave(jnp.int32(0), jnp.int32(0), 0, noskip=True)
    fire(jnp.int32(2))
    run_wave(jnp.int32(1), jnp.int32(0), 1, noskip=True)
    fire(jnp.int32(3))

    def pair(k, c):
        run_wave(2 * k, k, 0)
        fire(2 * k + 2)
        run_wave(2 * k + 1, k, 1)
        fire(2 * k + 3)
        return c

    lax.fori_loop(1, NWAVES // 2, pair, 0)

    for buf in range(2):
        wb, wv, s = bufs[buf]
        pltpu.make_async_copy(
            wv, out.at[plsc.Indices(wb, ignored_value=-1)], ssem).wait()


@jax.jit
def kernel(query_ids, model_ids, query_table, model_table):
    return _build_kernel()(query_ids.astype(jnp.int32),
                           model_ids.astype(jnp.int32),
                           query_table.T,
                           model_table.reshape(NMFLAT))
